# baseline (speedup 1.0000x reference)
import functools

import jax
import jax.numpy as jnp
from jax import lax
from jax.experimental import pallas as pl
from jax.experimental.pallas import tpu as pltpu
from jax.experimental.pallas import tpu_sc as plsc

_N = 10000
_E = 320000
_HID = 128
_NCONV = 3
_BE = 4000

_NW = 32
_NSLICE = 2
_ES = _E // _NSLICE


def _make_gather(e):
    epw = e // _NW
    ch = 200
    nch = epw // ch
    assert epw % ch == 0 and ch % 8 == 0 and nch >= 4

    def _gather_body(h_hbm, src_hbm, dst_hbm, hs_out, hd_out,
                     sidx, didx, sr0, sr1, dr0, dr1, sem_g, sem_w):
        wid = lax.axis_index("s") * 2 + lax.axis_index("c")
        base = wid * epw
        pltpu.sync_copy(src_hbm.at[pl.ds(base, epw)], sidx)
        pltpu.sync_copy(dst_hbm.at[pl.ds(base, epw)], didx)

        srows = (sr0, sr1)
        drows = (dr0, dr1)

        def chunk(j, b, drain):
            off = j * ch
            if drain:
                pltpu.make_async_copy(hs_out.at[pl.ds(base, ch)], srows[b], sem_w).wait()
                pltpu.make_async_copy(hd_out.at[pl.ds(base, ch)], drows[b], sem_w).wait()
            cs = pltpu.async_copy(h_hbm.at[sidx.at[pl.ds(off, ch)]], srows[b], sem_g)
            cd = pltpu.async_copy(h_hbm.at[didx.at[pl.ds(off, ch)]], drows[b], sem_g)
            cs.wait()
            cd.wait()
            pltpu.async_copy(srows[b], hs_out.at[pl.ds(base + off, ch)], sem_w)
            pltpu.async_copy(drows[b], hd_out.at[pl.ds(base + off, ch)], sem_w)

        chunk(0, 0, False)
        chunk(1, 1, False)

        def body(g, c):
            chunk(2 * g, 0, True)
            chunk(2 * g + 1, 1, True)
            return c

        lax.fori_loop(1, nch // 2, body, 0)
        if nch % 2 == 1:
            chunk(nch - 1, 0, True)
        pltpu.make_async_copy(hs_out.at[pl.ds(base, ch)], sr0, sem_w).wait()
        pltpu.make_async_copy(hd_out.at[pl.ds(base, ch)], dr0, sem_w).wait()
        pltpu.make_async_copy(hs_out.at[pl.ds(base, ch)], sr1, sem_w).wait()
        pltpu.make_async_copy(hd_out.at[pl.ds(base, ch)], dr1, sem_w).wait()

    return functools.partial(
        pl.kernel,
        mesh=plsc.VectorSubcoreMesh(core_axis_name="c", subcore_axis_name="s"),
        out_type=[
            jax.ShapeDtypeStruct((e, _HID), jnp.float32),
            jax.ShapeDtypeStruct((e, _HID), jnp.float32),
        ],
        scratch_types=[
            pltpu.VMEM((epw,), jnp.int32),
            pltpu.VMEM((epw,), jnp.int32),
            pltpu.VMEM((ch, _HID), jnp.float32),
            pltpu.VMEM((ch, _HID), jnp.float32),
            pltpu.VMEM((ch, _HID), jnp.float32),
            pltpu.VMEM((ch, _HID), jnp.float32),
            pltpu.SemaphoreType.DMA,
            pltpu.SemaphoreType.DMA,
        ],
    )(_gather_body)


_gather2 = _make_gather(_ES)


def _msg_body(hs_ref, hd_ref, at_ref, ws_ref, wm_ref, bs_ref, bm_ref, o_ref):
    z256 = jnp.concatenate([hs_ref[...], hd_ref[...]], axis=1)
    a16 = at_ref[...]
    ys = jnp.dot(z256, ws_ref[:2 * _HID]) + jnp.dot(a16, ws_ref[2 * _HID:]) + bs_ref[...]
    ym = jnp.dot(z256, wm_ref[:2 * _HID]) + jnp.dot(a16, wm_ref[2 * _HID:]) + bm_ref[...]
    o_ref[...] = jax.nn.sigmoid(ys) * jax.nn.softplus(ym)


def _msg(hs, hd, attr, ws_t, wm_t, bsi, bmi):
    e = hs.shape[0]
    zdim = ws_t.shape[0]
    return pl.pallas_call(
        _msg_body,
        grid=(e // _BE,),
        in_specs=[
            pl.BlockSpec((_BE, _HID), lambda i: (i, 0)),
            pl.BlockSpec((_BE, _HID), lambda i: (i, 0)),
            pl.BlockSpec((_BE, zdim - 2 * _HID), lambda i: (i, 0)),
            pl.BlockSpec((zdim, _HID), lambda i: (0, 0)),
            pl.BlockSpec((zdim, _HID), lambda i: (0, 0)),
            pl.BlockSpec((1, _HID), lambda i: (0, 0)),
            pl.BlockSpec((1, _HID), lambda i: (0, 0)),
        ],
        out_specs=pl.BlockSpec((_BE, _HID), lambda i: (i, 0)),
        out_shape=jax.ShapeDtypeStruct((e, _HID), jnp.float32),
    )(hs, hd, attr, ws_t, wm_t, bsi.reshape(1, _HID), bmi.reshape(1, _HID))


def kernel(x, edge_index, edge_attr, W_emb, b_emb, Ws, bs, Wm, bm, gamma,
           beta, W_fc, b_fc, W_out, b_out):
    src = edge_index[0]
    dst = edge_index[1]
    h = x @ W_emb.T + b_emb
    for i in range(_NCONV):
        parts = []
        for k in range(_NSLICE):
            sl = slice(k * _ES, (k + 1) * _ES)
            hs, hd = _gather2(h, src[sl], dst[sl])
            parts.append(_msg(hs, hd, edge_attr[sl], Ws[i].T, Wm[i].T,
                              bs[i], bm[i]))
        msg = jnp.concatenate(parts, axis=0) if _NSLICE > 1 else parts[0]
        h = h + jax.ops.segment_sum(msg, dst, num_segments=_N)
        mean = jnp.mean(h, axis=0)
        var = jnp.var(h, axis=0)
        h = (h - mean) / jnp.sqrt(var + 1e-5) * gamma[i] + beta[i]
    graph_feat = jnp.sum(h, axis=0, keepdims=True)
    graph_feat = graph_feat @ W_fc.T + b_fc
    out = graph_feat @ W_out.T + b_out
    return out


# speedup vs baseline: 1.0794x; 1.0794x over previous
import functools

import jax
import jax.numpy as jnp
from jax import lax
from jax.experimental import pallas as pl
from jax.experimental.pallas import tpu as pltpu
from jax.experimental.pallas import tpu_sc as plsc

_N = 10000
_E = 320000
_HID = 128
_NCONV = 3
_BE = 4000

_NW = 32
_NSLICE = 1
_ES = _E // _NSLICE


def _make_gather(e):
    epw = e // _NW
    ch = 200
    nch = epw // ch
    assert epw % ch == 0 and ch % 8 == 0 and nch >= 4

    def _gather_body(h_hbm, src_hbm, dst_hbm, hs_out, hd_out,
                     sidx, didx, sr0, sr1, dr0, dr1, sem_g, sem_w):
        wid = lax.axis_index("s") * 2 + lax.axis_index("c")
        base = wid * epw
        pltpu.sync_copy(src_hbm.at[pl.ds(base, epw)], sidx)
        pltpu.sync_copy(dst_hbm.at[pl.ds(base, epw)], didx)

        srows = (sr0, sr1)
        drows = (dr0, dr1)

        def chunk(j, b, drain):
            off = j * ch
            if drain:
                pltpu.make_async_copy(hs_out.at[pl.ds(base, ch)], srows[b], sem_w).wait()
                pltpu.make_async_copy(hd_out.at[pl.ds(base, ch)], drows[b], sem_w).wait()
            cs = pltpu.async_copy(h_hbm.at[sidx.at[pl.ds(off, ch)]], srows[b], sem_g)
            cd = pltpu.async_copy(h_hbm.at[didx.at[pl.ds(off, ch)]], drows[b], sem_g)
            cs.wait()
            cd.wait()
            pltpu.async_copy(srows[b], hs_out.at[pl.ds(base + off, ch)], sem_w)
            pltpu.async_copy(drows[b], hd_out.at[pl.ds(base + off, ch)], sem_w)

        chunk(0, 0, False)
        chunk(1, 1, False)

        def body(g, c):
            chunk(2 * g, 0, True)
            chunk(2 * g + 1, 1, True)
            return c

        lax.fori_loop(1, nch // 2, body, 0)
        if nch % 2 == 1:
            chunk(nch - 1, 0, True)
        pltpu.make_async_copy(hs_out.at[pl.ds(base, ch)], sr0, sem_w).wait()
        pltpu.make_async_copy(hd_out.at[pl.ds(base, ch)], dr0, sem_w).wait()
        pltpu.make_async_copy(hs_out.at[pl.ds(base, ch)], sr1, sem_w).wait()
        pltpu.make_async_copy(hd_out.at[pl.ds(base, ch)], dr1, sem_w).wait()

    return functools.partial(
        pl.kernel,
        mesh=plsc.VectorSubcoreMesh(core_axis_name="c", subcore_axis_name="s"),
        out_type=[
            jax.ShapeDtypeStruct((e, _HID), jnp.float32),
            jax.ShapeDtypeStruct((e, _HID), jnp.float32),
        ],
        scratch_types=[
            pltpu.VMEM((epw,), jnp.int32),
            pltpu.VMEM((epw,), jnp.int32),
            pltpu.VMEM((ch, _HID), jnp.float32),
            pltpu.VMEM((ch, _HID), jnp.float32),
            pltpu.VMEM((ch, _HID), jnp.float32),
            pltpu.VMEM((ch, _HID), jnp.float32),
            pltpu.SemaphoreType.DMA,
            pltpu.SemaphoreType.DMA,
        ],
    )(_gather_body)


_gather2 = _make_gather(_ES)


def _msg_body(hs_ref, hd_ref, at_ref, ws_ref, wm_ref, bs_ref, bm_ref, o_ref):
    z256 = jnp.concatenate([hs_ref[...], hd_ref[...]], axis=1)
    a16 = at_ref[...]
    ys = jnp.dot(z256, ws_ref[:2 * _HID]) + jnp.dot(a16, ws_ref[2 * _HID:]) + bs_ref[...]
    ym = jnp.dot(z256, wm_ref[:2 * _HID]) + jnp.dot(a16, wm_ref[2 * _HID:]) + bm_ref[...]
    o_ref[...] = jax.nn.sigmoid(ys) * jax.nn.softplus(ym)


def _msg(hs, hd, attr, ws_t, wm_t, bsi, bmi):
    e = hs.shape[0]
    zdim = ws_t.shape[0]
    return pl.pallas_call(
        _msg_body,
        grid=(e // _BE,),
        in_specs=[
            pl.BlockSpec((_BE, _HID), lambda i: (i, 0)),
            pl.BlockSpec((_BE, _HID), lambda i: (i, 0)),
            pl.BlockSpec((_BE, zdim - 2 * _HID), lambda i: (i, 0)),
            pl.BlockSpec((zdim, _HID), lambda i: (0, 0)),
            pl.BlockSpec((zdim, _HID), lambda i: (0, 0)),
            pl.BlockSpec((1, _HID), lambda i: (0, 0)),
            pl.BlockSpec((1, _HID), lambda i: (0, 0)),
        ],
        out_specs=pl.BlockSpec((_BE, _HID), lambda i: (i, 0)),
        out_shape=jax.ShapeDtypeStruct((e, _HID), jnp.float32),
    )(hs, hd, attr, ws_t, wm_t, bsi.reshape(1, _HID), bmi.reshape(1, _HID))


def kernel(x, edge_index, edge_attr, W_emb, b_emb, Ws, bs, Wm, bm, gamma,
           beta, W_fc, b_fc, W_out, b_out):
    src = edge_index[0]
    dst = edge_index[1]
    h = x @ W_emb.T + b_emb
    for i in range(_NCONV):
        parts = []
        for k in range(_NSLICE):
            sl = slice(k * _ES, (k + 1) * _ES)
            hs, hd = _gather2(h, src[sl], dst[sl])
            parts.append(_msg(hs, hd, edge_attr[sl], Ws[i].T, Wm[i].T,
                              bs[i], bm[i]))
        msg = jnp.concatenate(parts, axis=0) if _NSLICE > 1 else parts[0]
        h = h + jax.ops.segment_sum(msg, dst, num_segments=_N)
        mean = jnp.mean(h, axis=0)
        var = jnp.var(h, axis=0)
        h = (h - mean) / jnp.sqrt(var + 1e-5) * gamma[i] + beta[i]
    graph_feat = jnp.sum(h, axis=0, keepdims=True)
    graph_feat = graph_feat @ W_fc.T + b_fc
    out = graph_feat @ W_out.T + b_out
    return out


# speedup vs baseline: 1.1035x; 1.0223x over previous
import functools

import jax
import jax.numpy as jnp
from jax import lax
from jax.experimental import pallas as pl
from jax.experimental.pallas import tpu as pltpu
from jax.experimental.pallas import tpu_sc as plsc

_N = 10000
_E = 320000
_HID = 128
_NCONV = 3
_BE = 8000

_NW = 32
_NSLICE = 1
_ES = _E // _NSLICE


def _make_gather(e):
    epw = e // _NW
    ch = 200
    nch = epw // ch
    assert epw % ch == 0 and ch % 8 == 0 and nch >= 4

    def _gather_body(h_hbm, src_hbm, dst_hbm, hs_out, hd_out,
                     sidx, didx, sr0, sr1, dr0, dr1, sem_g, sem_w):
        wid = lax.axis_index("s") * 2 + lax.axis_index("c")
        base = wid * epw
        pltpu.sync_copy(src_hbm.at[pl.ds(base, epw)], sidx)
        pltpu.sync_copy(dst_hbm.at[pl.ds(base, epw)], didx)

        srows = (sr0, sr1)
        drows = (dr0, dr1)

        def chunk(j, b, drain):
            off = j * ch
            if drain:
                pltpu.make_async_copy(hs_out.at[pl.ds(base, ch)], srows[b], sem_w).wait()
                pltpu.make_async_copy(hd_out.at[pl.ds(base, ch)], drows[b], sem_w).wait()
            cs = pltpu.async_copy(h_hbm.at[sidx.at[pl.ds(off, ch)]], srows[b], sem_g)
            cd = pltpu.async_copy(h_hbm.at[didx.at[pl.ds(off, ch)]], drows[b], sem_g)
            cs.wait()
            cd.wait()
            pltpu.async_copy(srows[b], hs_out.at[pl.ds(base + off, ch)], sem_w)
            pltpu.async_copy(drows[b], hd_out.at[pl.ds(base + off, ch)], sem_w)

        chunk(0, 0, False)
        chunk(1, 1, False)

        def body(g, c):
            chunk(2 * g, 0, True)
            chunk(2 * g + 1, 1, True)
            return c

        lax.fori_loop(1, nch // 2, body, 0)
        if nch % 2 == 1:
            chunk(nch - 1, 0, True)
        pltpu.make_async_copy(hs_out.at[pl.ds(base, ch)], sr0, sem_w).wait()
        pltpu.make_async_copy(hd_out.at[pl.ds(base, ch)], dr0, sem_w).wait()
        pltpu.make_async_copy(hs_out.at[pl.ds(base, ch)], sr1, sem_w).wait()
        pltpu.make_async_copy(hd_out.at[pl.ds(base, ch)], dr1, sem_w).wait()

    return functools.partial(
        pl.kernel,
        mesh=plsc.VectorSubcoreMesh(core_axis_name="c", subcore_axis_name="s"),
        out_type=[
            jax.ShapeDtypeStruct((e, _HID), jnp.float32),
            jax.ShapeDtypeStruct((e, _HID), jnp.float32),
        ],
        scratch_types=[
            pltpu.VMEM((epw,), jnp.int32),
            pltpu.VMEM((epw,), jnp.int32),
            pltpu.VMEM((ch, _HID), jnp.float32),
            pltpu.VMEM((ch, _HID), jnp.float32),
            pltpu.VMEM((ch, _HID), jnp.float32),
            pltpu.VMEM((ch, _HID), jnp.float32),
            pltpu.SemaphoreType.DMA,
            pltpu.SemaphoreType.DMA,
        ],
    )(_gather_body)


_gather2 = _make_gather(_ES)


def _msg_body(hs_ref, hd_ref, at_ref, ws_ref, wm_ref, bs_ref, bm_ref, o_ref):
    z256 = jnp.concatenate([hs_ref[...], hd_ref[...]], axis=1)
    a16 = at_ref[...]
    ys = jnp.dot(z256, ws_ref[:2 * _HID]) + jnp.dot(a16, ws_ref[2 * _HID:]) + bs_ref[...]
    ym = jnp.dot(z256, wm_ref[:2 * _HID]) + jnp.dot(a16, wm_ref[2 * _HID:]) + bm_ref[...]
    o_ref[...] = jax.nn.sigmoid(ys) * jax.nn.softplus(ym)


def _msg(hs, hd, attr, ws_t, wm_t, bsi, bmi):
    e = hs.shape[0]
    zdim = ws_t.shape[0]
    return pl.pallas_call(
        _msg_body,
        grid=(e // _BE,),
        in_specs=[
            pl.BlockSpec((_BE, _HID), lambda i: (i, 0)),
            pl.BlockSpec((_BE, _HID), lambda i: (i, 0)),
            pl.BlockSpec((_BE, zdim - 2 * _HID), lambda i: (i, 0)),
            pl.BlockSpec((zdim, _HID), lambda i: (0, 0)),
            pl.BlockSpec((zdim, _HID), lambda i: (0, 0)),
            pl.BlockSpec((1, _HID), lambda i: (0, 0)),
            pl.BlockSpec((1, _HID), lambda i: (0, 0)),
        ],
        out_specs=pl.BlockSpec((_BE, _HID), lambda i: (i, 0)),
        out_shape=jax.ShapeDtypeStruct((e, _HID), jnp.float32),
    )(hs, hd, attr, ws_t, wm_t, bsi.reshape(1, _HID), bmi.reshape(1, _HID))


def kernel(x, edge_index, edge_attr, W_emb, b_emb, Ws, bs, Wm, bm, gamma,
           beta, W_fc, b_fc, W_out, b_out):
    src = edge_index[0]
    dst = edge_index[1]
    h = x @ W_emb.T + b_emb
    for i in range(_NCONV):
        parts = []
        for k in range(_NSLICE):
            sl = slice(k * _ES, (k + 1) * _ES)
            hs, hd = _gather2(h, src[sl], dst[sl])
            parts.append(_msg(hs, hd, edge_attr[sl], Ws[i].T, Wm[i].T,
                              bs[i], bm[i]))
        msg = jnp.concatenate(parts, axis=0) if _NSLICE > 1 else parts[0]
        h = h + jax.ops.segment_sum(msg, dst, num_segments=_N)
        mean = jnp.mean(h, axis=0)
        var = jnp.var(h, axis=0)
        h = (h - mean) / jnp.sqrt(var + 1e-5) * gamma[i] + beta[i]
    graph_feat = jnp.sum(h, axis=0, keepdims=True)
    graph_feat = graph_feat @ W_fc.T + b_fc
    out = graph_feat @ W_out.T + b_out
    return out


# speedup vs baseline: 1.1074x; 1.0036x over previous
import functools

import jax
import jax.numpy as jnp
from jax import lax
from jax.experimental import pallas as pl
from jax.experimental.pallas import tpu as pltpu
from jax.experimental.pallas import tpu_sc as plsc

_N = 10000
_E = 320000
_HID = 128
_NCONV = 3
_BE = 10000

_NW = 32
_NSLICE = 1
_ES = _E // _NSLICE


def _make_gather(e):
    epw = e // _NW
    ch = 200
    nch = epw // ch
    assert epw % ch == 0 and ch % 8 == 0 and nch >= 4

    def _gather_body(h_hbm, src_hbm, dst_hbm, hs_out, hd_out,
                     sidx, didx, sr0, sr1, dr0, dr1, sem_g, sem_w):
        wid = lax.axis_index("s") * 2 + lax.axis_index("c")
        base = wid * epw
        pltpu.sync_copy(src_hbm.at[pl.ds(base, epw)], sidx)
        pltpu.sync_copy(dst_hbm.at[pl.ds(base, epw)], didx)

        srows = (sr0, sr1)
        drows = (dr0, dr1)

        def chunk(j, b, drain):
            off = j * ch
            if drain:
                pltpu.make_async_copy(hs_out.at[pl.ds(base, ch)], srows[b], sem_w).wait()
                pltpu.make_async_copy(hd_out.at[pl.ds(base, ch)], drows[b], sem_w).wait()
            cs = pltpu.async_copy(h_hbm.at[sidx.at[pl.ds(off, ch)]], srows[b], sem_g)
            cd = pltpu.async_copy(h_hbm.at[didx.at[pl.ds(off, ch)]], drows[b], sem_g)
            cs.wait()
            cd.wait()
            pltpu.async_copy(srows[b], hs_out.at[pl.ds(base + off, ch)], sem_w)
            pltpu.async_copy(drows[b], hd_out.at[pl.ds(base + off, ch)], sem_w)

        chunk(0, 0, False)
        chunk(1, 1, False)

        def body(g, c):
            chunk(2 * g, 0, True)
            chunk(2 * g + 1, 1, True)
            return c

        lax.fori_loop(1, nch // 2, body, 0)
        if nch % 2 == 1:
            chunk(nch - 1, 0, True)
        pltpu.make_async_copy(hs_out.at[pl.ds(base, ch)], sr0, sem_w).wait()
        pltpu.make_async_copy(hd_out.at[pl.ds(base, ch)], dr0, sem_w).wait()
        pltpu.make_async_copy(hs_out.at[pl.ds(base, ch)], sr1, sem_w).wait()
        pltpu.make_async_copy(hd_out.at[pl.ds(base, ch)], dr1, sem_w).wait()

    return functools.partial(
        pl.kernel,
        mesh=plsc.VectorSubcoreMesh(core_axis_name="c", subcore_axis_name="s"),
        out_type=[
            jax.ShapeDtypeStruct((e, _HID), jnp.float32),
            jax.ShapeDtypeStruct((e, _HID), jnp.float32),
        ],
        scratch_types=[
            pltpu.VMEM((epw,), jnp.int32),
            pltpu.VMEM((epw,), jnp.int32),
            pltpu.VMEM((ch, _HID), jnp.float32),
            pltpu.VMEM((ch, _HID), jnp.float32),
            pltpu.VMEM((ch, _HID), jnp.float32),
            pltpu.VMEM((ch, _HID), jnp.float32),
            pltpu.SemaphoreType.DMA,
            pltpu.SemaphoreType.DMA,
        ],
    )(_gather_body)


_gather2 = _make_gather(_ES)


def _msg_body(hs_ref, hd_ref, at_ref, ws_ref, wm_ref, bs_ref, bm_ref, o_ref):
    z256 = jnp.concatenate([hs_ref[...], hd_ref[...]], axis=1)
    a16 = at_ref[...]
    ys = jnp.dot(z256, ws_ref[:2 * _HID]) + jnp.dot(a16, ws_ref[2 * _HID:]) + bs_ref[...]
    ym = jnp.dot(z256, wm_ref[:2 * _HID]) + jnp.dot(a16, wm_ref[2 * _HID:]) + bm_ref[...]
    o_ref[...] = jax.nn.sigmoid(ys) * jax.nn.softplus(ym)


def _msg(hs, hd, attr, ws_t, wm_t, bsi, bmi):
    e = hs.shape[0]
    zdim = ws_t.shape[0]
    return pl.pallas_call(
        _msg_body,
        grid=(e // _BE,),
        in_specs=[
            pl.BlockSpec((_BE, _HID), lambda i: (i, 0)),
            pl.BlockSpec((_BE, _HID), lambda i: (i, 0)),
            pl.BlockSpec((_BE, zdim - 2 * _HID), lambda i: (i, 0)),
            pl.BlockSpec((zdim, _HID), lambda i: (0, 0)),
            pl.BlockSpec((zdim, _HID), lambda i: (0, 0)),
            pl.BlockSpec((1, _HID), lambda i: (0, 0)),
            pl.BlockSpec((1, _HID), lambda i: (0, 0)),
        ],
        out_specs=pl.BlockSpec((_BE, _HID), lambda i: (i, 0)),
        out_shape=jax.ShapeDtypeStruct((e, _HID), jnp.float32),
    )(hs, hd, attr, ws_t, wm_t, bsi.reshape(1, _HID), bmi.reshape(1, _HID))


def kernel(x, edge_index, edge_attr, W_emb, b_emb, Ws, bs, Wm, bm, gamma,
           beta, W_fc, b_fc, W_out, b_out):
    src = edge_index[0]
    dst = edge_index[1]
    h = x @ W_emb.T + b_emb
    for i in range(_NCONV):
        parts = []
        for k in range(_NSLICE):
            sl = slice(k * _ES, (k + 1) * _ES)
            hs, hd = _gather2(h, src[sl], dst[sl])
            parts.append(_msg(hs, hd, edge_attr[sl], Ws[i].T, Wm[i].T,
                              bs[i], bm[i]))
        msg = jnp.concatenate(parts, axis=0) if _NSLICE > 1 else parts[0]
        h = h + jax.ops.segment_sum(msg, dst, num_segments=_N)
        mean = jnp.mean(h, axis=0)
        var = jnp.var(h, axis=0)
        h = (h - mean) / jnp.sqrt(var + 1e-5) * gamma[i] + beta[i]
    graph_feat = jnp.sum(h, axis=0, keepdims=True)
    graph_feat = graph_feat @ W_fc.T + b_fc
    out = graph_feat @ W_out.T + b_out
    return out
